# baseline (speedup 1.0000x reference)
import jax
import jax.numpy as jnp
from jax.experimental import pallas as pl
from jax.experimental.pallas import tpu as pltpu

_B, _N, _F, _H, _OUT, _K = 32, 256, 512, 512, 8, 5


def _dot(a, b):
    return jax.lax.dot(a, b, preferred_element_type=jnp.float32)


def _dot_hi(a, b):
    return jax.lax.dot(a, b, preferred_element_type=jnp.float32)


def _gcn_body(A_ref, x_ref,
              Ws0, bs0, Ws1, bs1, Ws2, bs2,
              Wf0, bf0, Wf1, bf1, Wf2, bf2,
              es_ref, ef_ref, gs_ref, gf_ref):
    A_s = A_ref[0, 0]
    A_f = A_ref[0, 1]
    x = x_ref[0]
    xs = x
    acc_s = jnp.zeros((_N, _H), jnp.float32)
    for (W, b) in ((Ws0, bs0), (Ws1, bs1), (Ws2, bs2)):
        xs = jax.nn.relu(_dot_hi(A_s, _dot_hi(xs, W[...])) + b[...])
        acc_s = acc_s + xs
    xf = x
    acc_f = jnp.zeros((_N, _H), jnp.float32)
    for (W, b) in ((Wf0, bf0), (Wf1, bf1), (Wf2, bf2)):
        xf = jax.nn.relu(_dot_hi(A_f, _dot_hi(xf, W[...])) + b[...])
        acc_f = acc_f + xf
    E_s = acc_s * (1.0 / 3.0)
    E_f = acc_f * (1.0 / 3.0)
    es_ref[0] = E_s
    ef_ref[0] = E_f
    gs_ref[0] = jnp.mean(E_s, axis=0, keepdims=True)
    gf_ref[0] = jnp.mean(E_f, axis=0, keepdims=True)


def _knn_body(gs_ref, gf_ref,
              msc_row_ref, msc_col_ref, mfc_row_ref, mfc_col_ref,
              idx1_ref, idx2_ref):
    r_iota = jax.lax.broadcasted_iota(jnp.int32, (_B, _B), 0)
    c_iota = jax.lax.broadcasted_iota(jnp.int32, (_B, _B), 1)
    eye_b = r_iota == c_iota

    def topk_idx(g, mask_row, mask_col):
        diff = g[:, None, :] - g[None, :, :]
        d = jnp.sum(diff * diff, axis=-1)
        bad = eye_b | (mask_row[...] > 0.5)
        d = jnp.where(bad, jnp.inf, d)
        taken = jnp.zeros((_B, _B), jnp.bool_)
        idx_cols = []
        for _ in range(_K):
            d_eff = jnp.where(taken, jnp.inf, d)
            m = jnp.min(d_eff, axis=1, keepdims=True)
            cand = jnp.where((d_eff <= m) & (~taken), c_iota, _B)
            amin = jnp.min(cand, axis=1, keepdims=True)
            idx_cols.append(amin)
            taken = taken | (c_iota == amin)
        idx = jnp.concatenate(idx_cols, axis=1)
        r_bk = jax.lax.broadcasted_iota(jnp.int32, (_B, _K), 0)
        null_col = mask_col[...] > 0.5
        return jnp.where(null_col, idx, r_bk)

    gs = gs_ref[...].reshape(_B, _H)
    gf = gf_ref[...].reshape(_B, _H)
    idx1_ref[...] = topk_idx(gf, msc_row_ref, msc_col_ref)
    idx2_ref[...] = topk_idx(gs, mfc_row_ref, mfc_col_ref)


def _gather_body(i1_ref, i2_ref, m1_ref, m2_ref,
                 es_ref, ef_ref, w1w_ref, w1b_ref, w2w_ref, w2b_ref,
                 attn_ref, ow_ref, ob_ref, out_ref, acc1_ref, acc2_ref):
    b = pl.program_id(0)
    k = pl.program_id(1)

    @pl.when(k == 0)
    def _init():
        acc1_ref[...] = es_ref[0]
        acc2_ref[...] = ef_ref[0]

    @pl.when(k > 0)
    def _accum():
        acc1_ref[...] += es_ref[0]
        acc2_ref[...] += ef_ref[0]

    @pl.when(k == _K - 1)
    def _tail():
        null1 = m1_ref[b] > 0
        null2 = m2_ref[b] > 0
        Eb1 = jnp.where(null1, acc1_ref[...] * (1.0 / _K), es_ref[0])
        Eb2 = jnp.where(null2, acc2_ref[...] * (1.0 / _K), ef_ref[0])
        t1 = _dot(Eb1, w1w_ref[...]) + w1b_ref[...]
        t2 = _dot(Eb2, w2w_ref[...]) + w2b_ref[...]
        z1 = _dot(t1, attn_ref[...])
        z2 = _dot(t2, attn_ref[...])
        g1 = jnp.where(z1 >= 0, z1, 0.3 * z1)
        g2 = jnp.where(z2 >= 0, z2, 0.3 * z2)
        m = jnp.maximum(g1, g2)
        x1 = jnp.exp(g1 - m)
        x2 = jnp.exp(g2 - m)
        a1 = x1 / (x1 + x2)
        a2 = x2 / (x1 + x2)
        combo = a1 * Eb1 + a2 * Eb2
        pooled = jnp.mean(combo, axis=0, keepdims=True)
        out_ref[pl.ds(b, 1), :] = _dot(pooled, ow_ref[...]) + ob_ref[...]


def kernel(A_batch, feature, no_sc_idx, no_fc_idx,
           W_sc0, b_sc0, W_sc1, b_sc1, W_sc2, b_sc2,
           W_fc0, b_fc0, W_fc1, b_fc1, W_fc2, b_fc2,
           w1_w, w1_b, w2_w, w2_b, attention, out_w, out_b):
    f32 = jnp.float32
    msc = no_sc_idx.astype(f32)
    mfc = no_fc_idx.astype(f32)

    const = lambda shape: pl.BlockSpec(shape, lambda b: tuple(0 for _ in shape))
    k1 = pl.pallas_call(
        _gcn_body,
        grid=(_B,),
        in_specs=[
            pl.BlockSpec((1, 2, _N, _N), lambda b: (b, 0, 0, 0)),
            pl.BlockSpec((1, _N, _F), lambda b: (b, 0, 0)),
            const((_F, _H)), const((1, _H)),
            const((_H, _H)), const((1, _H)),
            const((_H, _H)), const((1, _H)),
            const((_F, _H)), const((1, _H)),
            const((_H, _H)), const((1, _H)),
            const((_H, _H)), const((1, _H)),
        ],
        out_specs=[
            pl.BlockSpec((1, _N, _H), lambda b: (b, 0, 0)),
            pl.BlockSpec((1, _N, _H), lambda b: (b, 0, 0)),
            pl.BlockSpec((1, 1, _H), lambda b: (b, 0, 0)),
            pl.BlockSpec((1, 1, _H), lambda b: (b, 0, 0)),
        ],
        out_shape=[
            jax.ShapeDtypeStruct((_B, _N, _H), f32),
            jax.ShapeDtypeStruct((_B, _N, _H), f32),
            jax.ShapeDtypeStruct((_B, 1, _H), f32),
            jax.ShapeDtypeStruct((_B, 1, _H), f32),
        ],
    )
    es, ef, gs, gf = k1(
        A_batch, feature,
        W_sc0, b_sc0.reshape(1, _H), W_sc1, b_sc1.reshape(1, _H),
        W_sc2, b_sc2.reshape(1, _H),
        W_fc0, b_fc0.reshape(1, _H), W_fc1, b_fc1.reshape(1, _H),
        W_fc2, b_fc2.reshape(1, _H))

    k2 = pl.pallas_call(
        _knn_body,
        out_shape=[
            jax.ShapeDtypeStruct((_B, _K), jnp.int32),
            jax.ShapeDtypeStruct((_B, _K), jnp.int32),
        ],
    )
    idx1, idx2 = k2(gs, gf,
                    msc.reshape(1, _B), msc.reshape(_B, 1),
                    mfc.reshape(1, _B), mfc.reshape(_B, 1))

    k3 = pl.pallas_call(
        _gather_body,
        grid_spec=pltpu.PrefetchScalarGridSpec(
            num_scalar_prefetch=4,
            grid=(_B, _K),
            in_specs=[
                pl.BlockSpec((1, _N, _H),
                             lambda b, k, i1, i2, m1, m2: (i1[b, k], 0, 0)),
                pl.BlockSpec((1, _N, _H),
                             lambda b, k, i1, i2, m1, m2: (i2[b, k], 0, 0)),
                pl.BlockSpec((_H, _H), lambda b, k, *_: (0, 0)),
                pl.BlockSpec((1, _H), lambda b, k, *_: (0, 0)),
                pl.BlockSpec((_H, _H), lambda b, k, *_: (0, 0)),
                pl.BlockSpec((1, _H), lambda b, k, *_: (0, 0)),
                pl.BlockSpec((_H, 1), lambda b, k, *_: (0, 0)),
                pl.BlockSpec((_H, _OUT), lambda b, k, *_: (0, 0)),
                pl.BlockSpec((1, _OUT), lambda b, k, *_: (0, 0)),
            ],
            out_specs=pl.BlockSpec((_B, _OUT), lambda b, k, *_: (0, 0)),
            scratch_shapes=[
                pltpu.VMEM((_N, _H), jnp.float32),
                pltpu.VMEM((_N, _H), jnp.float32),
            ],
        ),
        out_shape=jax.ShapeDtypeStruct((_B, _OUT), f32),
    )
    return k3(idx1, idx2,
              no_sc_idx.astype(jnp.int32), no_fc_idx.astype(jnp.int32),
              es, ef, w1_w, w1_b.reshape(1, _H), w2_w, w2_b.reshape(1, _H),
              attention, out_w, out_b.reshape(1, _OUT))


# speedup vs baseline: 1.0349x; 1.0349x over previous
import jax
import jax.numpy as jnp
from jax.experimental import pallas as pl
from jax.experimental.pallas import tpu as pltpu

_B, _N, _F, _H, _OUT, _K = 32, 256, 512, 512, 8, 5


def _dot(a, b):
    return jax.lax.dot(a, b, preferred_element_type=jnp.float32)


def _dot_hi(a, b):
    return jax.lax.dot(a, b, preferred_element_type=jnp.float32)


_G1 = 2


def _gcn_body(A_ref, x_ref,
              Ws0, bs0, Ws1, bs1, Ws2, bs2,
              Wf0, bf0, Wf1, bf1, Wf2, bf2,
              es_ref, ef_ref, gs_ref, gf_ref):
    for g in range(_G1):
        A_s = A_ref[g, 0]
        A_f = A_ref[g, 1]
        x = x_ref[g]
        xs = x
        acc_s = jnp.zeros((_N, _H), jnp.float32)
        for (W, b) in ((Ws0, bs0), (Ws1, bs1), (Ws2, bs2)):
            xs = jax.nn.relu(_dot_hi(A_s, _dot_hi(xs, W[...])) + b[...])
            acc_s = acc_s + xs
        xf = x
        acc_f = jnp.zeros((_N, _H), jnp.float32)
        for (W, b) in ((Wf0, bf0), (Wf1, bf1), (Wf2, bf2)):
            xf = jax.nn.relu(_dot_hi(A_f, _dot_hi(xf, W[...])) + b[...])
            acc_f = acc_f + xf
        E_s = acc_s * (1.0 / 3.0)
        E_f = acc_f * (1.0 / 3.0)
        es_ref[g] = E_s
        ef_ref[g] = E_f
        gs_ref[g] = jnp.mean(E_s, axis=0, keepdims=True)
        gf_ref[g] = jnp.mean(E_f, axis=0, keepdims=True)


def _knn_body(gs_ref, gf_ref,
              msc_row_ref, msc_col_ref, mfc_row_ref, mfc_col_ref,
              idx1_ref, idx2_ref):
    r_iota = jax.lax.broadcasted_iota(jnp.int32, (_B, _B), 0)
    c_iota = jax.lax.broadcasted_iota(jnp.int32, (_B, _B), 1)
    eye_b = r_iota == c_iota

    def topk_idx(g, mask_row, mask_col):
        diff = g[:, None, :] - g[None, :, :]
        d = jnp.sum(diff * diff, axis=-1)
        bad = eye_b | (mask_row[...] > 0.5)
        d = jnp.where(bad, jnp.inf, d)
        taken = jnp.zeros((_B, _B), jnp.bool_)
        idx_cols = []
        for _ in range(_K):
            d_eff = jnp.where(taken, jnp.inf, d)
            m = jnp.min(d_eff, axis=1, keepdims=True)
            cand = jnp.where((d_eff <= m) & (~taken), c_iota, _B)
            amin = jnp.min(cand, axis=1, keepdims=True)
            idx_cols.append(amin)
            taken = taken | (c_iota == amin)
        idx = jnp.concatenate(idx_cols, axis=1)
        r_bk = jax.lax.broadcasted_iota(jnp.int32, (_B, _K), 0)
        null_col = mask_col[...] > 0.5
        return jnp.where(null_col, idx, r_bk)

    gs = gs_ref[...].reshape(_B, _H)
    gf = gf_ref[...].reshape(_B, _H)
    idx1_ref[...] = topk_idx(gf, msc_row_ref, msc_col_ref)
    idx2_ref[...] = topk_idx(gs, mfc_row_ref, mfc_col_ref)


def _gather_body(i1_ref, i2_ref, m1_ref, m2_ref,
                 es_ref, ef_ref, w1w_ref, w1b_ref, w2w_ref, w2b_ref,
                 attn_ref, ow_ref, ob_ref, out_ref, acc1_ref, acc2_ref):
    b = pl.program_id(0)
    k = pl.program_id(1)

    @pl.when(k == 0)
    def _init():
        acc1_ref[...] = es_ref[0]
        acc2_ref[...] = ef_ref[0]

    @pl.when(k > 0)
    def _accum():
        acc1_ref[...] += es_ref[0]
        acc2_ref[...] += ef_ref[0]

    @pl.when(k == _K - 1)
    def _tail():
        null1 = m1_ref[b] > 0
        null2 = m2_ref[b] > 0
        Eb1 = jnp.where(null1, acc1_ref[...] * (1.0 / _K), es_ref[0])
        Eb2 = jnp.where(null2, acc2_ref[...] * (1.0 / _K), ef_ref[0])
        t1 = _dot(Eb1, w1w_ref[...]) + w1b_ref[...]
        t2 = _dot(Eb2, w2w_ref[...]) + w2b_ref[...]
        z1 = _dot(t1, attn_ref[...])
        z2 = _dot(t2, attn_ref[...])
        g1 = jnp.where(z1 >= 0, z1, 0.3 * z1)
        g2 = jnp.where(z2 >= 0, z2, 0.3 * z2)
        m = jnp.maximum(g1, g2)
        x1 = jnp.exp(g1 - m)
        x2 = jnp.exp(g2 - m)
        a1 = x1 / (x1 + x2)
        a2 = x2 / (x1 + x2)
        combo = a1 * Eb1 + a2 * Eb2
        pooled = jnp.mean(combo, axis=0, keepdims=True)
        out_ref[pl.ds(b, 1), :] = _dot(pooled, ow_ref[...]) + ob_ref[...]


def kernel(A_batch, feature, no_sc_idx, no_fc_idx,
           W_sc0, b_sc0, W_sc1, b_sc1, W_sc2, b_sc2,
           W_fc0, b_fc0, W_fc1, b_fc1, W_fc2, b_fc2,
           w1_w, w1_b, w2_w, w2_b, attention, out_w, out_b):
    f32 = jnp.float32
    msc = no_sc_idx.astype(f32)
    mfc = no_fc_idx.astype(f32)

    const = lambda shape: pl.BlockSpec(shape, lambda b: tuple(0 for _ in shape))
    k1 = pl.pallas_call(
        _gcn_body,
        grid=(_B // _G1,),
        in_specs=[
            pl.BlockSpec((_G1, 2, _N, _N), lambda b: (b, 0, 0, 0)),
            pl.BlockSpec((_G1, _N, _F), lambda b: (b, 0, 0)),
            const((_F, _H)), const((1, _H)),
            const((_H, _H)), const((1, _H)),
            const((_H, _H)), const((1, _H)),
            const((_F, _H)), const((1, _H)),
            const((_H, _H)), const((1, _H)),
            const((_H, _H)), const((1, _H)),
        ],
        out_specs=[
            pl.BlockSpec((_G1, _N, _H), lambda b: (b, 0, 0)),
            pl.BlockSpec((_G1, _N, _H), lambda b: (b, 0, 0)),
            pl.BlockSpec((_G1, 1, _H), lambda b: (b, 0, 0)),
            pl.BlockSpec((_G1, 1, _H), lambda b: (b, 0, 0)),
        ],
        out_shape=[
            jax.ShapeDtypeStruct((_B, _N, _H), f32),
            jax.ShapeDtypeStruct((_B, _N, _H), f32),
            jax.ShapeDtypeStruct((_B, 1, _H), f32),
            jax.ShapeDtypeStruct((_B, 1, _H), f32),
        ],
    )
    es, ef, gs, gf = k1(
        A_batch, feature,
        W_sc0, b_sc0.reshape(1, _H), W_sc1, b_sc1.reshape(1, _H),
        W_sc2, b_sc2.reshape(1, _H),
        W_fc0, b_fc0.reshape(1, _H), W_fc1, b_fc1.reshape(1, _H),
        W_fc2, b_fc2.reshape(1, _H))

    k2 = pl.pallas_call(
        _knn_body,
        out_shape=[
            jax.ShapeDtypeStruct((_B, _K), jnp.int32),
            jax.ShapeDtypeStruct((_B, _K), jnp.int32),
        ],
    )
    idx1, idx2 = k2(gs, gf,
                    msc.reshape(1, _B), msc.reshape(_B, 1),
                    mfc.reshape(1, _B), mfc.reshape(_B, 1))

    k3 = pl.pallas_call(
        _gather_body,
        grid_spec=pltpu.PrefetchScalarGridSpec(
            num_scalar_prefetch=4,
            grid=(_B, _K),
            in_specs=[
                pl.BlockSpec((1, _N, _H),
                             lambda b, k, i1, i2, m1, m2: (i1[b, k], 0, 0)),
                pl.BlockSpec((1, _N, _H),
                             lambda b, k, i1, i2, m1, m2: (i2[b, k], 0, 0)),
                pl.BlockSpec((_H, _H), lambda b, k, *_: (0, 0)),
                pl.BlockSpec((1, _H), lambda b, k, *_: (0, 0)),
                pl.BlockSpec((_H, _H), lambda b, k, *_: (0, 0)),
                pl.BlockSpec((1, _H), lambda b, k, *_: (0, 0)),
                pl.BlockSpec((_H, 1), lambda b, k, *_: (0, 0)),
                pl.BlockSpec((_H, _OUT), lambda b, k, *_: (0, 0)),
                pl.BlockSpec((1, _OUT), lambda b, k, *_: (0, 0)),
            ],
            out_specs=pl.BlockSpec((_B, _OUT), lambda b, k, *_: (0, 0)),
            scratch_shapes=[
                pltpu.VMEM((_N, _H), jnp.float32),
                pltpu.VMEM((_N, _H), jnp.float32),
            ],
        ),
        out_shape=jax.ShapeDtypeStruct((_B, _OUT), f32),
    )
    return k3(idx1, idx2,
              no_sc_idx.astype(jnp.int32), no_fc_idx.astype(jnp.int32),
              es, ef, w1_w, w1_b.reshape(1, _H), w2_w, w2_b.reshape(1, _H),
              attention, out_w, out_b.reshape(1, _OUT))


# speedup vs baseline: 1.0564x; 1.0208x over previous
import jax
import jax.numpy as jnp
from jax.experimental import pallas as pl
from jax.experimental.pallas import tpu as pltpu

_B, _N, _F, _H, _OUT, _K = 32, 256, 512, 512, 8, 5


def _dot(a, b):
    return jax.lax.dot(a, b, preferred_element_type=jnp.float32)


def _dot_hi(a, b):
    return jax.lax.dot(a, b, preferred_element_type=jnp.float32)


_G1 = 4


def _gcn_body(A_ref, x_ref,
              Ws0, bs0, Ws1, bs1, Ws2, bs2,
              Wf0, bf0, Wf1, bf1, Wf2, bf2,
              es_ref, ef_ref, gs_ref, gf_ref):
    for g in range(_G1):
        A_s = A_ref[g, 0]
        A_f = A_ref[g, 1]
        x = x_ref[g]
        xs = x
        acc_s = jnp.zeros((_N, _H), jnp.float32)
        for (W, b) in ((Ws0, bs0), (Ws1, bs1), (Ws2, bs2)):
            xs = jax.nn.relu(_dot_hi(A_s, _dot_hi(xs, W[...])) + b[...])
            acc_s = acc_s + xs
        xf = x
        acc_f = jnp.zeros((_N, _H), jnp.float32)
        for (W, b) in ((Wf0, bf0), (Wf1, bf1), (Wf2, bf2)):
            xf = jax.nn.relu(_dot_hi(A_f, _dot_hi(xf, W[...])) + b[...])
            acc_f = acc_f + xf
        E_s = acc_s * (1.0 / 3.0)
        E_f = acc_f * (1.0 / 3.0)
        es_ref[g] = E_s
        ef_ref[g] = E_f
        gs_ref[g] = jnp.mean(E_s, axis=0, keepdims=True)
        gf_ref[g] = jnp.mean(E_f, axis=0, keepdims=True)


def _knn_body(gs_ref, gf_ref,
              msc_row_ref, msc_col_ref, mfc_row_ref, mfc_col_ref,
              idx1_ref, idx2_ref):
    r_iota = jax.lax.broadcasted_iota(jnp.int32, (_B, _B), 0)
    c_iota = jax.lax.broadcasted_iota(jnp.int32, (_B, _B), 1)
    eye_b = r_iota == c_iota

    def topk_idx(g, mask_row, mask_col):
        diff = g[:, None, :] - g[None, :, :]
        d = jnp.sum(diff * diff, axis=-1)
        bad = eye_b | (mask_row[...] > 0.5)
        d = jnp.where(bad, jnp.inf, d)
        taken = jnp.zeros((_B, _B), jnp.bool_)
        idx_cols = []
        for _ in range(_K):
            d_eff = jnp.where(taken, jnp.inf, d)
            m = jnp.min(d_eff, axis=1, keepdims=True)
            cand = jnp.where((d_eff <= m) & (~taken), c_iota, _B)
            amin = jnp.min(cand, axis=1, keepdims=True)
            idx_cols.append(amin)
            taken = taken | (c_iota == amin)
        idx = jnp.concatenate(idx_cols, axis=1)
        r_bk = jax.lax.broadcasted_iota(jnp.int32, (_B, _K), 0)
        null_col = mask_col[...] > 0.5
        return jnp.where(null_col, idx, r_bk)

    gs = gs_ref[...].reshape(_B, _H)
    gf = gf_ref[...].reshape(_B, _H)
    idx1_ref[...] = topk_idx(gf, msc_row_ref, msc_col_ref)
    idx2_ref[...] = topk_idx(gs, mfc_row_ref, mfc_col_ref)


def _gather_body(i1_ref, i2_ref, m1_ref, m2_ref,
                 es_ref, ef_ref, w1w_ref, w1b_ref, w2w_ref, w2b_ref,
                 attn_ref, ow_ref, ob_ref, out_ref, acc1_ref, acc2_ref):
    b = pl.program_id(0)
    k = pl.program_id(1)

    @pl.when(k == 0)
    def _init():
        acc1_ref[...] = es_ref[0]
        acc2_ref[...] = ef_ref[0]

    @pl.when(k > 0)
    def _accum():
        acc1_ref[...] += es_ref[0]
        acc2_ref[...] += ef_ref[0]

    @pl.when(k == _K - 1)
    def _tail():
        null1 = m1_ref[b] > 0
        null2 = m2_ref[b] > 0
        Eb1 = jnp.where(null1, acc1_ref[...] * (1.0 / _K), es_ref[0])
        Eb2 = jnp.where(null2, acc2_ref[...] * (1.0 / _K), ef_ref[0])
        t1 = _dot(Eb1, w1w_ref[...]) + w1b_ref[...]
        t2 = _dot(Eb2, w2w_ref[...]) + w2b_ref[...]
        z1 = _dot(t1, attn_ref[...])
        z2 = _dot(t2, attn_ref[...])
        g1 = jnp.where(z1 >= 0, z1, 0.3 * z1)
        g2 = jnp.where(z2 >= 0, z2, 0.3 * z2)
        m = jnp.maximum(g1, g2)
        x1 = jnp.exp(g1 - m)
        x2 = jnp.exp(g2 - m)
        a1 = x1 / (x1 + x2)
        a2 = x2 / (x1 + x2)
        combo = a1 * Eb1 + a2 * Eb2
        pooled = jnp.mean(combo, axis=0, keepdims=True)
        out_ref[pl.ds(b, 1), :] = _dot(pooled, ow_ref[...]) + ob_ref[...]


def kernel(A_batch, feature, no_sc_idx, no_fc_idx,
           W_sc0, b_sc0, W_sc1, b_sc1, W_sc2, b_sc2,
           W_fc0, b_fc0, W_fc1, b_fc1, W_fc2, b_fc2,
           w1_w, w1_b, w2_w, w2_b, attention, out_w, out_b):
    f32 = jnp.float32
    msc = no_sc_idx.astype(f32)
    mfc = no_fc_idx.astype(f32)

    const = lambda shape: pl.BlockSpec(shape, lambda b: tuple(0 for _ in shape))
    k1 = pl.pallas_call(
        _gcn_body,
        grid=(_B // _G1,),
        in_specs=[
            pl.BlockSpec((_G1, 2, _N, _N), lambda b: (b, 0, 0, 0)),
            pl.BlockSpec((_G1, _N, _F), lambda b: (b, 0, 0)),
            const((_F, _H)), const((1, _H)),
            const((_H, _H)), const((1, _H)),
            const((_H, _H)), const((1, _H)),
            const((_F, _H)), const((1, _H)),
            const((_H, _H)), const((1, _H)),
            const((_H, _H)), const((1, _H)),
        ],
        out_specs=[
            pl.BlockSpec((_G1, _N, _H), lambda b: (b, 0, 0)),
            pl.BlockSpec((_G1, _N, _H), lambda b: (b, 0, 0)),
            pl.BlockSpec((_G1, 1, _H), lambda b: (b, 0, 0)),
            pl.BlockSpec((_G1, 1, _H), lambda b: (b, 0, 0)),
        ],
        out_shape=[
            jax.ShapeDtypeStruct((_B, _N, _H), f32),
            jax.ShapeDtypeStruct((_B, _N, _H), f32),
            jax.ShapeDtypeStruct((_B, 1, _H), f32),
            jax.ShapeDtypeStruct((_B, 1, _H), f32),
        ],
    )
    es, ef, gs, gf = k1(
        A_batch, feature,
        W_sc0, b_sc0.reshape(1, _H), W_sc1, b_sc1.reshape(1, _H),
        W_sc2, b_sc2.reshape(1, _H),
        W_fc0, b_fc0.reshape(1, _H), W_fc1, b_fc1.reshape(1, _H),
        W_fc2, b_fc2.reshape(1, _H))

    k2 = pl.pallas_call(
        _knn_body,
        out_shape=[
            jax.ShapeDtypeStruct((_B, _K), jnp.int32),
            jax.ShapeDtypeStruct((_B, _K), jnp.int32),
        ],
    )
    idx1, idx2 = k2(gs, gf,
                    msc.reshape(1, _B), msc.reshape(_B, 1),
                    mfc.reshape(1, _B), mfc.reshape(_B, 1))

    k3 = pl.pallas_call(
        _gather_body,
        grid_spec=pltpu.PrefetchScalarGridSpec(
            num_scalar_prefetch=4,
            grid=(_B, _K),
            in_specs=[
                pl.BlockSpec((1, _N, _H),
                             lambda b, k, i1, i2, m1, m2: (i1[b, k], 0, 0)),
                pl.BlockSpec((1, _N, _H),
                             lambda b, k, i1, i2, m1, m2: (i2[b, k], 0, 0)),
                pl.BlockSpec((_H, _H), lambda b, k, *_: (0, 0)),
                pl.BlockSpec((1, _H), lambda b, k, *_: (0, 0)),
                pl.BlockSpec((_H, _H), lambda b, k, *_: (0, 0)),
                pl.BlockSpec((1, _H), lambda b, k, *_: (0, 0)),
                pl.BlockSpec((_H, 1), lambda b, k, *_: (0, 0)),
                pl.BlockSpec((_H, _OUT), lambda b, k, *_: (0, 0)),
                pl.BlockSpec((1, _OUT), lambda b, k, *_: (0, 0)),
            ],
            out_specs=pl.BlockSpec((_B, _OUT), lambda b, k, *_: (0, 0)),
            scratch_shapes=[
                pltpu.VMEM((_N, _H), jnp.float32),
                pltpu.VMEM((_N, _H), jnp.float32),
            ],
        ),
        out_shape=jax.ShapeDtypeStruct((_B, _OUT), f32),
    )
    return k3(idx1, idx2,
              no_sc_idx.astype(jnp.int32), no_fc_idx.astype(jnp.int32),
              es, ef, w1_w, w1_b.reshape(1, _H), w2_w, w2_b.reshape(1, _H),
              attention, out_w, out_b.reshape(1, _OUT))
